# baseline (speedup 1.0000x reference)
import functools

import jax
import jax.numpy as jnp
from jax import lax
from jax.experimental import pallas as pl
from jax.experimental.pallas import tpu as pltpu
from jax.experimental.pallas import tpu_sc as plsc

N_EMBED = 8192
EMBED_DIM = 256
COMMITMENT = 0.25
M_TILE = 512

NC, NS, L = 2, 16, 16
NW = NC * NS
CHUNK = 128
HALF = NW * CHUNK
BINS_PER_S = N_EMBED // NS


def _dist_argmin_body(x_ref, e_ref, idx_ref, et_ref, se_ref):
    m = pl.program_id(0)
    x = x_ref[...]
    e = e_ref[...]
    if et_ref is not None:
        slab = N_EMBED // pl.num_programs(0)
        et_ref[...] = e_ref[:, pl.ds(m * slab, slab)].T

    @pl.when(m == 0)
    def _se():
        se_ref[...] = jnp.sum(e * e, axis=0, keepdims=True)

    sx = jnp.sum(x * x, axis=1, keepdims=True)
    se = se_ref[...]
    mm2 = jnp.dot(x * 2.0, e, preferred_element_type=jnp.float32)
    dist = sx - mm2 + se
    tmin = jnp.min(dist, axis=1, keepdims=True)
    cols = jax.lax.broadcasted_iota(jnp.int32, (1, N_EMBED), 1).astype(jnp.float32)
    targ = jnp.min(jnp.where(dist == tmin, cols, jnp.float32(1e9)),
                   axis=1, keepdims=True)
    idx_ref[...] = targ.astype(jnp.int32)


def _dist_argmin(flat, embed, emit_et):
    t = flat.shape[0]
    grid = t // M_TILE
    slab = N_EMBED // grid
    out_specs = [pl.BlockSpec((M_TILE, 1), lambda m: (m, 0))]
    out_shape = [jax.ShapeDtypeStruct((t, 1), jnp.int32)]
    if emit_et:
        out_specs.append(pl.BlockSpec((slab, EMBED_DIM), lambda m: (m, 0)))
        out_shape.append(jax.ShapeDtypeStruct((N_EMBED, EMBED_DIM), jnp.float32))
        body = _dist_argmin_body
    else:
        def body(x_ref, e_ref, idx_ref, se_ref):
            _dist_argmin_body(x_ref, e_ref, idx_ref, None, se_ref)
    return pl.pallas_call(
        body,
        grid=(grid,),
        in_specs=[
            pl.BlockSpec((M_TILE, EMBED_DIM), lambda m: (m, 0)),
            pl.BlockSpec((EMBED_DIM, N_EMBED), lambda m: (0, 0)),
        ],
        out_specs=out_specs,
        out_shape=out_shape,
        scratch_shapes=[
            pltpu.VMEM((1, N_EMBED), jnp.float32),
        ],
    )(flat, embed)


def _sc_gather_hist(embed_t, idx32):
    mesh = plsc.VectorSubcoreMesh(core_axis_name="c", subcore_axis_name="s")

    def body(et_hbm, idx_hbm, quant_hbm, hist_hbm,
             idx_v, rows_v, ones_v, zero_v, hist_sh, sem):
        cid = lax.axis_index("c")
        sid = lax.axis_index("s")
        wid = sid * NC + cid

        pltpu.sync_copy(idx_hbm.at[pl.ds(wid, 1)], idx_v)

        for i in range(BINS_PER_S // L):
            zero_v[pl.ds(i * L, L)] = jnp.zeros((L,), jnp.float32)
        pltpu.sync_copy(zero_v, hist_sh.at[pl.ds(sid * BINS_PER_S, BINS_PER_S)])

        pltpu.async_copy(et_hbm.at[idx_v.at[0]], rows_v, sem).wait()
        pltpu.sync_copy(rows_v, quant_hbm.at[pl.ds(wid * CHUNK, CHUNK)])

        for i in range(CHUNK // L):
            ones_v[pl.ds(i * L, L)] = jnp.full((L,), 1.0, jnp.float32)

        plsc.subcore_barrier()
        pltpu.sync_copy(ones_v, hist_sh.at[idx_v.at[0]], add=True)
        plsc.subcore_barrier()

        pltpu.sync_copy(hist_sh.at[pl.ds(sid * BINS_PER_S, BINS_PER_S)],
                        hist_hbm.at[cid, sid])

    f = pl.kernel(
        body,
        mesh=mesh,
        out_type=[
            jax.ShapeDtypeStruct((HALF, EMBED_DIM), jnp.float32),
            jax.ShapeDtypeStruct((NC, NS, BINS_PER_S), jnp.float32),
        ],
        scratch_types=[
            pltpu.VMEM((1, CHUNK), jnp.int32),
            pltpu.VMEM((CHUNK, EMBED_DIM), jnp.float32),
            pltpu.VMEM((CHUNK,), jnp.float32),
            pltpu.VMEM((BINS_PER_S,), jnp.float32),
            pltpu.VMEM_SHARED((N_EMBED,), jnp.float32),
            pltpu.SemaphoreType.DMA,
        ],
    )
    return f(embed_t, idx32)


def _loss_body(hist_ref, lik_ref, loss_ref):
    h = hist_ref[...]
    counts = (h[0] + h[1]) + (h[2] + h[3])
    lik = counts * (1.0 / N_EMBED)
    lik_ref[...] = lik
    log_lik = jnp.log(lik + 1e-10)
    log_u = jnp.log(jnp.float32(1.0 / N_EMBED))
    s = jnp.sum((1.0 / N_EMBED) * (log_u - log_lik))
    loss_ref[...] = jnp.reshape(s / N_EMBED * COMMITMENT, (1, 1))


def _loss_tc(hist_part):
    return pl.pallas_call(
        _loss_body,
        out_shape=[
            jax.ShapeDtypeStruct((64, 128), jnp.float32),
            jax.ShapeDtypeStruct((1, 1), jnp.float32),
        ],
    )(hist_part)


def kernel(x, embed):
    flat = x.reshape(-1, EMBED_DIM)
    idx_a, embed_t = _dist_argmin(flat[:HALF], embed, emit_et=True)
    (idx_b,) = _dist_argmin(flat[HALF:], embed, emit_et=False)
    quant_a, hist_a = _sc_gather_hist(embed_t, idx_a.reshape(NW, CHUNK))
    quant_b, hist_b = _sc_gather_hist(embed_t, idx_b.reshape(NW, CHUNK))
    quantize = jnp.concatenate([quant_a, quant_b], axis=0).reshape(x.shape)
    hist4 = jnp.concatenate([hist_a, hist_b], axis=0)
    lik64, loss11 = _loss_tc(hist4.reshape(4, 64, 128))
    likelihoods = lik64.reshape(-1)
    quant_loss = loss11.reshape(())
    sigma = jnp.array(1e10, dtype=jnp.float32)
    return (quantize, quant_loss, embed_t, likelihoods, sigma)


# speedup vs baseline: 1.1401x; 1.1401x over previous
import functools

import jax
import jax.numpy as jnp
from jax import lax
from jax.experimental import pallas as pl
from jax.experimental.pallas import tpu as pltpu
from jax.experimental.pallas import tpu_sc as plsc

N_EMBED = 8192
EMBED_DIM = 256
COMMITMENT = 0.25
N_TILE = 512

NC, NS, L = 2, 16, 16
NW = NC * NS
B_PER_W = N_EMBED // NW
CHUNK = 128
BINS_PER_S = N_EMBED // NS


def _dist_argmin_body(x_ref, e_ref, idx_ref, et_ref, se_ref):
    m = pl.program_id(0)
    x = x_ref[...]
    e = e_ref[...]
    et_ref[...] = e_ref[:, pl.ds(m * M_TILE, M_TILE)].T

    @pl.when(m == 0)
    def _se():
        se_ref[...] = jnp.sum(e * e, axis=0, keepdims=True)

    sx = jnp.sum(x * x, axis=1, keepdims=True)
    se = se_ref[...]
    mm2 = jnp.dot(x * 2.0, e, preferred_element_type=jnp.float32)
    dist = sx - mm2 + se
    tmin = jnp.min(dist, axis=1, keepdims=True)
    cols = jax.lax.broadcasted_iota(jnp.int32, (1, N_EMBED), 1).astype(jnp.float32)
    targ = jnp.min(jnp.where(dist == tmin, cols, jnp.float32(1e9)),
                   axis=1, keepdims=True)
    idx_ref[...] = targ.astype(jnp.int32)


M_TILE = 512


def _dist_argmin(flat, embed):
    t = flat.shape[0]
    grid = t // M_TILE
    return pl.pallas_call(
        _dist_argmin_body,
        grid=(grid,),
        in_specs=[
            pl.BlockSpec((M_TILE, EMBED_DIM), lambda m: (m, 0)),
            pl.BlockSpec((EMBED_DIM, N_EMBED), lambda m: (0, 0)),
        ],
        out_specs=[
            pl.BlockSpec((M_TILE, 1), lambda m: (m, 0)),
            pl.BlockSpec((M_TILE, EMBED_DIM), lambda m: (m, 0)),
        ],
        out_shape=[
            jax.ShapeDtypeStruct((t, 1), jnp.int32),
            jax.ShapeDtypeStruct((N_EMBED, EMBED_DIM), jnp.float32),
        ],
        scratch_shapes=[
            pltpu.VMEM((1, N_EMBED), jnp.float32),
        ],
    )(flat, embed)


def _sc_gather_hist(embed_t, idx64):
    mesh = plsc.VectorSubcoreMesh(core_axis_name="c", subcore_axis_name="s")

    def body(et_hbm, idx_hbm, quant_hbm, hist_hbm,
             idx_v, rows_v, ones_v, zero_v, hist_sh, sem, sem2):
        cid = lax.axis_index("c")
        sid = lax.axis_index("s")
        wid = sid * NC + cid

        pltpu.sync_copy(idx_hbm.at[pl.ds(2 * wid, 2)], idx_v)

        for i in range(BINS_PER_S // L):
            zero_v[pl.ds(i * L, L)] = jnp.zeros((L,), jnp.float32)
        pltpu.sync_copy(zero_v, hist_sh.at[pl.ds(sid * BINS_PER_S, BINS_PER_S)])

        cp0 = pltpu.async_copy(et_hbm.at[idx_v.at[0]], rows_v.at[0], sem)
        cp1 = pltpu.async_copy(et_hbm.at[idx_v.at[1]], rows_v.at[1], sem)

        for i in range(CHUNK // L):
            ones_v[pl.ds(i * L, L)] = jnp.full((L,), 1.0, jnp.float32)

        plsc.subcore_barrier()
        pltpu.sync_copy(ones_v, hist_sh.at[idx_v.at[0]], add=True)
        pltpu.sync_copy(ones_v, hist_sh.at[idx_v.at[1]], add=True)

        cp0.wait()
        wb0 = pltpu.async_copy(rows_v.at[0], quant_hbm.at[2 * wid], sem2)
        cp1.wait()
        wb1 = pltpu.async_copy(rows_v.at[1], quant_hbm.at[2 * wid + 1], sem2)

        plsc.subcore_barrier()
        pltpu.sync_copy(hist_sh.at[pl.ds(sid * BINS_PER_S, BINS_PER_S)],
                        hist_hbm.at[cid, sid])
        wb0.wait()
        wb1.wait()

    f = pl.kernel(
        body,
        mesh=mesh,
        out_type=[
            jax.ShapeDtypeStruct((NW * 2, CHUNK, EMBED_DIM), jnp.float32),
            jax.ShapeDtypeStruct((NC, NS, BINS_PER_S), jnp.float32),
        ],
        scratch_types=[
            pltpu.VMEM((2, CHUNK), jnp.int32),
            pltpu.VMEM((2, CHUNK, EMBED_DIM), jnp.float32),
            pltpu.VMEM((CHUNK,), jnp.float32),
            pltpu.VMEM((BINS_PER_S,), jnp.float32),
            pltpu.VMEM_SHARED((N_EMBED,), jnp.float32),
            pltpu.SemaphoreType.DMA,
            pltpu.SemaphoreType.DMA,
        ],
    )
    return f(embed_t, idx64)


def _loss_body(hist_ref, lik_ref, loss_ref):
    h = hist_ref[...]
    counts = h[0] + h[1]
    lik = counts * (1.0 / N_EMBED)
    lik_ref[...] = lik
    log_lik = jnp.log(lik + 1e-10)
    log_u = jnp.log(jnp.float32(1.0 / N_EMBED))
    s = jnp.sum((1.0 / N_EMBED) * (log_u - log_lik))
    loss_ref[...] = jnp.reshape(s / N_EMBED * COMMITMENT, (1, 1))


def _loss_tc(hist_part):
    return pl.pallas_call(
        _loss_body,
        out_shape=[
            jax.ShapeDtypeStruct((64, 128), jnp.float32),
            jax.ShapeDtypeStruct((1, 1), jnp.float32),
        ],
    )(hist_part)


def kernel(x, embed):
    flat = x.reshape(-1, EMBED_DIM)
    idx2d, embed_t = _dist_argmin(flat, embed)
    idx64 = idx2d.reshape(NW * 2, CHUNK)
    quant64, hist_part = _sc_gather_hist(embed_t, idx64)
    quantize = quant64.reshape(x.shape)
    lik64, loss11 = _loss_tc(hist_part.reshape(NC, 64, 128))
    likelihoods = lik64.reshape(-1)
    quant_loss = loss11.reshape(())
    sigma = jnp.array(1e10, dtype=jnp.float32)
    return (quantize, quant_loss, embed_t, likelihoods, sigma)


# speedup vs baseline: 1.1530x; 1.0113x over previous
import functools

import jax
import jax.numpy as jnp
from jax import lax
from jax.experimental import pallas as pl
from jax.experimental.pallas import tpu as pltpu
from jax.experimental.pallas import tpu_sc as plsc

N_EMBED = 8192
EMBED_DIM = 256
COMMITMENT = 0.25
N_TILE = 512

NC, NS, L = 2, 16, 16
NW = NC * NS
B_PER_W = N_EMBED // NW
CHUNK = 128
BINS_PER_S = N_EMBED // NS


def _dist_argmin_body(x_ref, e_ref, idx_ref, et_ref, se_ref):
    m = pl.program_id(0)
    x = x_ref[...]
    e = e_ref[...]
    et_ref[...] = e_ref[:, pl.ds(m * M_TILE, M_TILE)].T

    @pl.when(m == 0)
    def _se():
        se_ref[...] = jnp.sum(e * e, axis=0, keepdims=True)

    sx = jnp.sum(x * x, axis=1, keepdims=True)
    se = se_ref[...]
    mm2 = jnp.dot(x * 2.0, e, preferred_element_type=jnp.float32)
    dist = sx - mm2 + se
    tmin = jnp.min(dist, axis=1, keepdims=True)
    cols = jax.lax.broadcasted_iota(jnp.int32, (1, N_EMBED), 1).astype(jnp.float32)
    targ = jnp.min(jnp.where(dist == tmin, cols, jnp.float32(1e9)),
                   axis=1, keepdims=True)
    idx_ref[...] = targ.astype(jnp.int32)


M_TILE = 512


def _dist_argmin(flat, embed):
    t = flat.shape[0]
    grid = t // M_TILE
    return pl.pallas_call(
        _dist_argmin_body,
        grid=(grid,),
        in_specs=[
            pl.BlockSpec((M_TILE, EMBED_DIM), lambda m: (m, 0)),
            pl.BlockSpec((EMBED_DIM, N_EMBED), lambda m: (0, 0)),
        ],
        out_specs=[
            pl.BlockSpec((M_TILE, 1), lambda m: (m, 0)),
            pl.BlockSpec((M_TILE, EMBED_DIM), lambda m: (m, 0)),
        ],
        out_shape=[
            jax.ShapeDtypeStruct((t, 1), jnp.int32),
            jax.ShapeDtypeStruct((N_EMBED, EMBED_DIM), jnp.float32),
        ],
        scratch_shapes=[
            pltpu.VMEM((1, N_EMBED), jnp.float32),
        ],
    )(flat, embed)


def _sc_gather_hist(embed_t, idx64):
    mesh = plsc.VectorSubcoreMesh(core_axis_name="c", subcore_axis_name="s")

    def body(et_hbm, idx_hbm, quant_hbm, hist_hbm,
             idx_v, rows_v, ones_v, zero_v, hist_sh, sem):
        cid = lax.axis_index("c")
        sid = lax.axis_index("s")
        wid = sid * NC + cid

        pltpu.sync_copy(idx_hbm.at[pl.ds(2 * wid, 2)], idx_v)

        for i in range(BINS_PER_S // L):
            zero_v[pl.ds(i * L, L)] = jnp.zeros((L,), jnp.float32)
        pltpu.sync_copy(zero_v, hist_sh.at[pl.ds(sid * BINS_PER_S, BINS_PER_S)])

        cp0 = pltpu.async_copy(et_hbm.at[idx_v.at[0]], rows_v.at[0], sem)
        cp1 = pltpu.async_copy(et_hbm.at[idx_v.at[1]], rows_v.at[1], sem)
        cp0.wait()
        cp1.wait()
        pltpu.sync_copy(rows_v, quant_hbm.at[pl.ds(2 * wid, 2)])

        for i in range(CHUNK // L):
            ones_v[pl.ds(i * L, L)] = jnp.full((L,), 1.0, jnp.float32)

        plsc.subcore_barrier()
        pltpu.sync_copy(ones_v, hist_sh.at[idx_v.at[0]], add=True)
        pltpu.sync_copy(ones_v, hist_sh.at[idx_v.at[1]], add=True)
        plsc.subcore_barrier()

        pltpu.sync_copy(hist_sh.at[pl.ds(sid * BINS_PER_S, BINS_PER_S)],
                        hist_hbm.at[cid, sid])

    f = pl.kernel(
        body,
        mesh=mesh,
        out_type=[
            jax.ShapeDtypeStruct((NW * 2, CHUNK, EMBED_DIM), jnp.float32),
            jax.ShapeDtypeStruct((NC, NS, BINS_PER_S), jnp.float32),
        ],
        scratch_types=[
            pltpu.VMEM((2, CHUNK), jnp.int32),
            pltpu.VMEM((2, CHUNK, EMBED_DIM), jnp.float32),
            pltpu.VMEM((CHUNK,), jnp.float32),
            pltpu.VMEM((BINS_PER_S,), jnp.float32),
            pltpu.VMEM_SHARED((N_EMBED,), jnp.float32),
            pltpu.SemaphoreType.DMA,
        ],
    )
    return f(embed_t, idx64)


def _loss_body(hist_ref, lik_ref, loss_ref):
    h = hist_ref[...]
    counts = h[0] + h[1]
    lik = counts * (1.0 / N_EMBED)
    lik_ref[...] = lik
    log_lik = jnp.log(lik + 1e-10)
    log_u = jnp.log(jnp.float32(1.0 / N_EMBED))
    s = jnp.sum((1.0 / N_EMBED) * (log_u - log_lik))
    loss_ref[...] = jnp.reshape(s / N_EMBED * COMMITMENT, (1, 1))


def _loss_tc(hist_part):
    return pl.pallas_call(
        _loss_body,
        out_shape=[
            jax.ShapeDtypeStruct((NS, BINS_PER_S), jnp.float32),
            jax.ShapeDtypeStruct((1, 1), jnp.float32),
        ],
    )(hist_part)


def kernel(x, embed):
    flat = x.reshape(-1, EMBED_DIM)
    idx2d, embed_t = _dist_argmin(flat, embed)
    idx64 = idx2d.reshape(NW * 2, CHUNK)
    quant64, hist_part = _sc_gather_hist(embed_t, idx64)
    quantize = quant64.reshape(x.shape)
    lik2, loss11 = _loss_tc(hist_part)
    likelihoods = lik2.reshape(-1)
    quant_loss = loss11.reshape(())
    sigma = jnp.array(1e10, dtype=jnp.float32)
    return (quantize, quant_loss, embed_t, likelihoods, sigma)


# speedup vs baseline: 1.1850x; 1.0278x over previous
import functools

import jax
import jax.numpy as jnp
from jax import lax
from jax.experimental import pallas as pl
from jax.experimental.pallas import tpu as pltpu
from jax.experimental.pallas import tpu_sc as plsc

N_EMBED = 8192
EMBED_DIM = 256
COMMITMENT = 0.25
N_TILE = 512

NC, NS, L = 2, 16, 16
NW = NC * NS
B_PER_W = N_EMBED // NW
CHUNK = 128
BINS_PER_S = N_EMBED // NS


def _dist_argmin_body(x_ref, e_ref, idx_ref, et_ref, se_ref):
    m = pl.program_id(0)
    x = x_ref[...]
    e = e_ref[...]
    et_ref[...] = e_ref[:, pl.ds(m * M_TILE, M_TILE)].T

    @pl.when(m == 0)
    def _se():
        se_ref[...] = jnp.sum(e * e, axis=0, keepdims=True)

    sx = jnp.sum(x * x, axis=1, keepdims=True)
    se = se_ref[...]
    mm2 = jnp.dot(x * 2.0, e, preferred_element_type=jnp.float32)
    dist = sx - mm2 + se
    tmin = jnp.min(dist, axis=1, keepdims=True)
    cols = jax.lax.broadcasted_iota(jnp.int32, (1, N_EMBED), 1).astype(jnp.float32)
    targ = jnp.min(jnp.where(dist == tmin, cols, jnp.float32(1e9)),
                   axis=1, keepdims=True)
    idx_ref[...] = targ.astype(jnp.int32).T.reshape(1, 1, M_TILE)


M_TILE = 512


def _dist_argmin(flat, embed):
    t = flat.shape[0]
    grid = t // M_TILE
    return pl.pallas_call(
        _dist_argmin_body,
        grid=(grid,),
        in_specs=[
            pl.BlockSpec((M_TILE, EMBED_DIM), lambda m: (m, 0)),
            pl.BlockSpec((EMBED_DIM, N_EMBED), lambda m: (0, 0)),
        ],
        out_specs=[
            pl.BlockSpec((1, 1, M_TILE), lambda m: (m, 0, 0)),
            pl.BlockSpec((M_TILE, EMBED_DIM), lambda m: (m, 0)),
        ],
        out_shape=[
            jax.ShapeDtypeStruct((t // M_TILE, 1, M_TILE), jnp.int32),
            jax.ShapeDtypeStruct((N_EMBED, EMBED_DIM), jnp.float32),
        ],
        scratch_shapes=[
            pltpu.VMEM((1, N_EMBED), jnp.float32),
        ],
    )(flat, embed)


def _sc_gather_hist(embed_t, idx64):
    mesh = plsc.VectorSubcoreMesh(core_axis_name="c", subcore_axis_name="s")

    def body(et_hbm, idx_hbm, quant_hbm, hist_hbm,
             idx_v, rows_v, ones_v, zero_v, hist_sh, sem):
        cid = lax.axis_index("c")
        sid = lax.axis_index("s")
        wid = sid * NC + cid

        pltpu.sync_copy(idx_hbm.at[wid // 2, 0, pl.ds((wid % 2) * 256, CHUNK)],
                        idx_v.at[0])
        pltpu.sync_copy(idx_hbm.at[wid // 2, 0, pl.ds((wid % 2) * 256 + CHUNK, CHUNK)],
                        idx_v.at[1])

        for i in range(BINS_PER_S // L):
            zero_v[pl.ds(i * L, L)] = jnp.zeros((L,), jnp.float32)
        pltpu.sync_copy(zero_v, hist_sh.at[pl.ds(sid * BINS_PER_S, BINS_PER_S)])

        cp0 = pltpu.async_copy(et_hbm.at[idx_v.at[0]], rows_v.at[0], sem)
        cp1 = pltpu.async_copy(et_hbm.at[idx_v.at[1]], rows_v.at[1], sem)
        cp0.wait()
        cp1.wait()
        pltpu.sync_copy(rows_v, quant_hbm.at[pl.ds(2 * wid, 2)])

        for i in range(CHUNK // L):
            ones_v[pl.ds(i * L, L)] = jnp.full((L,), 1.0, jnp.float32)

        plsc.subcore_barrier()
        pltpu.sync_copy(ones_v, hist_sh.at[idx_v.at[0]], add=True)
        pltpu.sync_copy(ones_v, hist_sh.at[idx_v.at[1]], add=True)
        plsc.subcore_barrier()

        pltpu.sync_copy(hist_sh.at[pl.ds(sid * BINS_PER_S, BINS_PER_S)],
                        hist_hbm.at[cid, sid])

    f = pl.kernel(
        body,
        mesh=mesh,
        out_type=[
            jax.ShapeDtypeStruct((NW * 2, CHUNK, EMBED_DIM), jnp.float32),
            jax.ShapeDtypeStruct((NC, NS, BINS_PER_S), jnp.float32),
        ],
        scratch_types=[
            pltpu.VMEM((2, CHUNK), jnp.int32),
            pltpu.VMEM((2, CHUNK, EMBED_DIM), jnp.float32),
            pltpu.VMEM((CHUNK,), jnp.float32),
            pltpu.VMEM((BINS_PER_S,), jnp.float32),
            pltpu.VMEM_SHARED((N_EMBED,), jnp.float32),
            pltpu.SemaphoreType.DMA,
        ],
    )
    return f(embed_t, idx64)


def _loss_body(hist_ref, lik_ref, loss_ref):
    h = hist_ref[...]
    counts = h[0] + h[1]
    lik = counts * (1.0 / N_EMBED)
    lik_ref[...] = lik
    log_lik = jnp.log(lik + 1e-10)
    log_u = jnp.log(jnp.float32(1.0 / N_EMBED))
    s = jnp.sum((1.0 / N_EMBED) * (log_u - log_lik))
    loss_ref[...] = jnp.reshape(s / N_EMBED * COMMITMENT, (1, 1))


def _loss_tc(hist_part):
    return pl.pallas_call(
        _loss_body,
        out_shape=[
            jax.ShapeDtypeStruct((NS, BINS_PER_S), jnp.float32),
            jax.ShapeDtypeStruct((1, 1), jnp.float32),
        ],
    )(hist_part)


def kernel(x, embed):
    flat = x.reshape(-1, EMBED_DIM)
    idx2d, embed_t = _dist_argmin(flat, embed)
    quant64, hist_part = _sc_gather_hist(embed_t, idx2d)
    quantize = quant64.reshape(x.shape)
    lik2, loss11 = _loss_tc(hist_part)
    likelihoods = lik2.reshape(-1)
    quant_loss = loss11.reshape(())
    sigma = jnp.array(1e10, dtype=jnp.float32)
    return (quantize, quant_loss, embed_t, likelihoods, sigma)
